# baseline (speedup 1.0000x reference)
import functools

import jax
import jax.numpy as jnp
from jax import lax
from jax.experimental import pallas as pl
from jax.experimental.pallas import tpu as pltpu
from jax.experimental.pallas import tpu_sc as plsc

NUM_EXPERTS = 64
TOP_K = 2
HIDDEN = 768
TOKEN_BLOCK = 2048
N_TOKENS = 4 * 8192
NW = 32
CHUNK = N_TOKENS // NW


def _gate_body(hs_ref, gw_ref, logits_ref):
    logits_t = lax.dot_general(
        gw_ref[...], hs_ref[...],
        dimension_numbers=(((1,), (1,)), ((), ())),
        preferred_element_type=jnp.float32,
    )
    logits_ref[...] = logits_t.T


@jax.jit
def _gate(hs2d, gw):
    n_tokens = hs2d.shape[0]
    grid = (n_tokens // TOKEN_BLOCK,)
    return pl.pallas_call(
        _gate_body,
        grid=grid,
        in_specs=[
            pl.BlockSpec((TOKEN_BLOCK, HIDDEN), lambda i: (i, 0)),
            pl.BlockSpec((NUM_EXPERTS, HIDDEN), lambda i: (0, 0)),
        ],
        out_specs=pl.BlockSpec((TOKEN_BLOCK, NUM_EXPERTS), lambda i: (i, 0)),
        out_shape=jax.ShapeDtypeStruct((n_tokens, NUM_EXPERTS), jnp.float32),
    )(hs2d, gw)


def _sc_topk_body(logits_hbm, w_hbm, e_hbm, lg_v, w_v, e_v):
    wid = lax.axis_index("s") * 2 + lax.axis_index("c")
    base = wid * CHUNK
    pltpu.sync_copy(logits_hbm.at[pl.ds(base * NUM_EXPERTS, CHUNK * NUM_EXPERTS)], lg_v)

    def group(g, carry):
        row = g * 16 + lax.iota(jnp.int32, 16)
        rowb = row * NUM_EXPERTS
        m0 = jnp.full((16,), -jnp.inf, jnp.float32)
        m1 = jnp.full((16,), -jnp.inf, jnp.float32)
        i0 = jnp.zeros((16,), jnp.int32)
        i1 = jnp.zeros((16,), jnp.int32)
        for e in range(NUM_EXPERTS):
            col = jnp.full((16,), e, jnp.int32)
            x = plsc.load_gather(lg_v, (rowb + e,))
            gt0 = x > m0
            gt1 = x > m1
            m1 = jnp.where(gt0, m0, jnp.where(gt1, x, m1))
            i1 = jnp.where(gt0, i0, jnp.where(gt1, col, i1))
            m0 = jnp.where(gt0, x, m0)
            i0 = jnp.where(gt0, col, i0)
        eh = jnp.exp(m1 - m0)
        w0 = 1.0 / (1.0 + eh)
        w1 = eh / (1.0 + eh)
        fi0 = row * 2
        fi1 = fi0 + 1
        plsc.store_scatter(w_v, (fi0,), w0)
        plsc.store_scatter(w_v, (fi1,), w1)
        plsc.store_scatter(e_v, (fi0,), i0)
        plsc.store_scatter(e_v, (fi1,), i1)
        return carry

    lax.fori_loop(0, CHUNK // 16, group, 0)
    pltpu.sync_copy(w_v, w_hbm.at[pl.ds(base * 2, CHUNK * 2)])
    pltpu.sync_copy(e_v, e_hbm.at[pl.ds(base * 2, CHUNK * 2)])


_sc_topk = functools.partial(
    pl.kernel,
    mesh=plsc.VectorSubcoreMesh(core_axis_name="c", subcore_axis_name="s"),
    compiler_params=pltpu.CompilerParams(needs_layout_passes=False),
    out_type=[
        jax.ShapeDtypeStruct((N_TOKENS * TOP_K,), jnp.float32),
        jax.ShapeDtypeStruct((N_TOKENS * TOP_K,), jnp.int32),
    ],
    scratch_types=[
        pltpu.VMEM((CHUNK * NUM_EXPERTS,), jnp.float32),
        pltpu.VMEM((CHUNK * TOP_K,), jnp.float32),
        pltpu.VMEM((CHUNK * TOP_K,), jnp.int32),
    ],
)(_sc_topk_body)


def kernel(hidden_states, gate_w):
    batch, seq, hidden = hidden_states.shape
    hs2d = hidden_states.reshape(batch * seq, hidden)
    logits = _gate(hs2d, gate_w)
    w_flat, e_flat = _sc_topk(logits.reshape(-1))
    return (
        w_flat.reshape(batch, seq, TOP_K),
        e_flat.reshape(batch, seq, TOP_K),
        logits.reshape(batch, seq, NUM_EXPERTS),
    )


# speedup vs baseline: 1.3711x; 1.3711x over previous
import functools

import jax
import jax.numpy as jnp
from jax import lax
from jax.experimental import pallas as pl
from jax.experimental.pallas import tpu as pltpu
from jax.experimental.pallas import tpu_sc as plsc

NUM_EXPERTS = 64
TOP_K = 2
HIDDEN = 768
TOKEN_BLOCK = 2048
N_TOKENS = 4 * 8192
NW = 32
CHUNK = N_TOKENS // NW


def _gate_body(hs_ref, gw_ref, logits_ref, w_ref, e_ref):
    logits_t = lax.dot_general(
        gw_ref[...], hs_ref[...],
        dimension_numbers=(((1,), (1,)), ((), ())),
        preferred_element_type=jnp.float32,
    )
    logits_ref[...] = logits_t.T

    t = logits_t.shape[1]
    eidx = lax.broadcasted_iota(jnp.int32, (NUM_EXPERTS, t), 0)
    neg_inf = jnp.float32(float("-inf"))

    m0 = jnp.max(logits_t, axis=0, keepdims=True)
    i0 = jnp.min(jnp.where(logits_t == m0, eidx, NUM_EXPERTS), axis=0, keepdims=True)
    masked = jnp.where(eidx == i0, neg_inf, logits_t)
    m1 = jnp.max(masked, axis=0, keepdims=True)
    i1 = jnp.min(jnp.where(masked == m1, eidx, NUM_EXPERTS), axis=0, keepdims=True)

    e = jnp.exp(m1 - m0)
    w0 = 1.0 / (1.0 + e)
    w1 = e / (1.0 + e)

    kidx = lax.broadcasted_iota(jnp.int32, (TOP_K, t), 0)
    w_ref[...] = jnp.where(kidx == 0, w0, w1)
    e_ref[...] = jnp.where(kidx == 0, i0, i1)


@jax.jit
def _gate(hs2d, gw):
    n_tokens = hs2d.shape[0]
    grid = (n_tokens // TOKEN_BLOCK,)
    return pl.pallas_call(
        _gate_body,
        grid=grid,
        in_specs=[
            pl.BlockSpec((TOKEN_BLOCK, HIDDEN), lambda i: (i, 0)),
            pl.BlockSpec((NUM_EXPERTS, HIDDEN), lambda i: (0, 0)),
        ],
        out_specs=[
            pl.BlockSpec((TOKEN_BLOCK, NUM_EXPERTS), lambda i: (i, 0)),
            pl.BlockSpec((TOP_K, TOKEN_BLOCK), lambda i: (0, i)),
            pl.BlockSpec((TOP_K, TOKEN_BLOCK), lambda i: (0, i)),
        ],
        out_shape=[
            jax.ShapeDtypeStruct((n_tokens, NUM_EXPERTS), jnp.float32),
            jax.ShapeDtypeStruct((TOP_K, n_tokens), jnp.float32),
            jax.ShapeDtypeStruct((TOP_K, n_tokens), jnp.int32),
        ],
    )(hs2d, gw)


def _sc_interleave_body(wt_hbm, et_hbm, w_hbm, e_hbm, wt_v, et_v, w_v, e_v):
    wid = lax.axis_index("s") * 2 + lax.axis_index("c")
    base = wid * CHUNK
    pltpu.sync_copy(wt_hbm.at[:, pl.ds(base, CHUNK)], wt_v)
    pltpu.sync_copy(et_hbm.at[:, pl.ds(base, CHUNK)], et_v)

    def group(g, carry):
        off = g * 16
        fi0 = (off + lax.iota(jnp.int32, 16)) * 2
        fi1 = fi0 + 1
        plsc.store_scatter(w_v, (fi0,), wt_v[0, pl.ds(off, 16)])
        plsc.store_scatter(w_v, (fi1,), wt_v[1, pl.ds(off, 16)])
        plsc.store_scatter(e_v, (fi0,), et_v[0, pl.ds(off, 16)])
        plsc.store_scatter(e_v, (fi1,), et_v[1, pl.ds(off, 16)])
        return carry

    lax.fori_loop(0, CHUNK // 16, group, 0)
    pltpu.sync_copy(w_v, w_hbm.at[pl.ds(base * 2, CHUNK * 2)])
    pltpu.sync_copy(e_v, e_hbm.at[pl.ds(base * 2, CHUNK * 2)])


_sc_interleave = functools.partial(
    pl.kernel,
    mesh=plsc.VectorSubcoreMesh(core_axis_name="c", subcore_axis_name="s"),
    compiler_params=pltpu.CompilerParams(needs_layout_passes=False),
    out_type=[
        jax.ShapeDtypeStruct((N_TOKENS * TOP_K,), jnp.float32),
        jax.ShapeDtypeStruct((N_TOKENS * TOP_K,), jnp.int32),
    ],
    scratch_types=[
        pltpu.VMEM((TOP_K, CHUNK), jnp.float32),
        pltpu.VMEM((TOP_K, CHUNK), jnp.int32),
        pltpu.VMEM((CHUNK * TOP_K,), jnp.float32),
        pltpu.VMEM((CHUNK * TOP_K,), jnp.int32),
    ],
)(_sc_interleave_body)


def kernel(hidden_states, gate_w):
    batch, seq, hidden = hidden_states.shape
    hs2d = hidden_states.reshape(batch * seq, hidden)
    logits, w_t, e_t = _gate(hs2d, gate_w)
    w_flat, e_flat = _sc_interleave(w_t, e_t)
    return (
        w_flat.reshape(batch, seq, TOP_K),
        e_flat.reshape(batch, seq, TOP_K),
        logits.reshape(batch, seq, NUM_EXPERTS),
    )


# speedup vs baseline: 2.5606x; 1.8676x over previous
import functools

import jax
import jax.numpy as jnp
from jax import lax
from jax.experimental import pallas as pl
from jax.experimental.pallas import tpu as pltpu

NUM_EXPERTS = 64
TOP_K = 2
HIDDEN = 768
TOKEN_BLOCK = 4096


def _router_body(hs_ref, gw_ref, logits_ref, w_ref, e_ref):
    logits_t = lax.dot_general(
        gw_ref[...], hs_ref[...],
        dimension_numbers=(((1,), (1,)), ((), ())),
        preferred_element_type=jnp.float32,
    )
    logits_ref[...] = logits_t.T

    t = logits_t.shape[1]
    eidx = lax.broadcasted_iota(jnp.int32, (NUM_EXPERTS, t), 0)
    neg_inf = jnp.float32(float("-inf"))

    m0 = jnp.max(logits_t, axis=0, keepdims=True)
    i0 = jnp.min(jnp.where(logits_t == m0, eidx, NUM_EXPERTS), axis=0, keepdims=True)
    masked = jnp.where(eidx == i0, neg_inf, logits_t)
    m1 = jnp.max(masked, axis=0, keepdims=True)
    i1 = jnp.min(jnp.where(masked == m1, eidx, NUM_EXPERTS), axis=0, keepdims=True)

    e = jnp.exp(m1 - m0)
    w0 = 1.0 / (1.0 + e)
    w1 = e / (1.0 + e)

    kidx = lax.broadcasted_iota(jnp.int32, (TOP_K, t), 0)
    w_ref[...] = jnp.where(kidx == 0, w0, w1)
    e_ref[...] = jnp.where(kidx == 0, i0, i1)


@jax.jit
def _router(hs2d, gw):
    n_tokens = hs2d.shape[0]
    grid = (n_tokens // TOKEN_BLOCK,)
    return pl.pallas_call(
        _router_body,
        grid=grid,
        in_specs=[
            pl.BlockSpec((TOKEN_BLOCK, HIDDEN), lambda i: (i, 0)),
            pl.BlockSpec((NUM_EXPERTS, HIDDEN), lambda i: (0, 0)),
        ],
        out_specs=[
            pl.BlockSpec((TOKEN_BLOCK, NUM_EXPERTS), lambda i: (i, 0)),
            pl.BlockSpec((TOP_K, TOKEN_BLOCK), lambda i: (0, i)),
            pl.BlockSpec((TOP_K, TOKEN_BLOCK), lambda i: (0, i)),
        ],
        out_shape=[
            jax.ShapeDtypeStruct((n_tokens, NUM_EXPERTS), jnp.float32),
            jax.ShapeDtypeStruct((TOP_K, n_tokens), jnp.float32),
            jax.ShapeDtypeStruct((TOP_K, n_tokens), jnp.int32),
        ],
    )(hs2d, gw)


def kernel(hidden_states, gate_w):
    batch, seq, hidden = hidden_states.shape
    hs2d = hidden_states.reshape(batch * seq, hidden)
    logits, weights_t, experts_t = _router(hs2d, gate_w)
    weights = weights_t.T.reshape(batch, seq, TOP_K)
    experts = experts_t.T.reshape(batch, seq, TOP_K)
    return weights, experts, logits.reshape(batch, seq, NUM_EXPERTS)
